# baseline (speedup 1.0000x reference)
import jax
import jax.numpy as jnp
from jax import lax
from jax.experimental import pallas as pl
from jax.experimental.pallas import tpu as pltpu
from jax.experimental.pallas import tpu_sc as plsc

_D = 64
_P = 4096
_T = 524288
_NC, _NS = 2, 16
_NW = _NC * _NS
_TW = _T // _NW
_C = 256
_NSUB = _C // 128
_NG = _C // 16
_CHUNKS = _TW // _C


def _sc_body(cand_idx_hbm, nb_idx_hbm, seg_hbm, table_hbm, s1_hbm, s2_hbm,
             cand_idx_v, nb_all, seg_all, idx2_0, idx2_1,
             rows_0, rows_1, cands_0, cands_1,
             acc1_v, acc2_v, sem0, sem1):
    wid = lax.axis_index("s") * _NC + lax.axis_index("c")
    base = wid * _TW

    idx2_b = (idx2_0, idx2_1)
    rows_b = (rows_0, rows_1)
    cands_b = (cands_0, cands_1)
    sem_b = (sem0, sem1)

    pltpu.sync_copy(cand_idx_hbm, cand_idx_v)
    pltpu.sync_copy(nb_idx_hbm.at[pl.ds(base, _TW)], nb_all)
    pltpu.sync_copy(seg_hbm.at[pl.ds(base, _TW)], seg_all)

    zeros16 = jnp.zeros((16,), jnp.float32)

    def zero_body(i, _):
        acc1_v[pl.ds(i * 16, 16)] = zeros16
        acc2_v[pl.ds(i * 16, 16)] = zeros16
        return 0

    lax.fori_loop(0, _P // 16, zero_body, 0, unroll=8)

    lane = lax.iota(jnp.int32, 16)

    def fire(ci, b):
        off = ci * _C
        idx2, rows, cands, sem = idx2_b[b], rows_b[b], cands_b[b], sem_b[b]

        def idx_body(g, _):
            sv = seg_all[pl.ds(off + g * 16, 16)]
            idx2[pl.ds(g * 16, 16)] = plsc.load_gather(cand_idx_v, [sv])
            return 0

        lax.fori_loop(0, _NG, idx_body, 0, unroll=_NG)
        for k in range(_NSUB):
            pltpu.async_copy(
                table_hbm.at[nb_all.at[pl.ds(off + k * 128, 128)]],
                rows.at[pl.ds(k * 128, 128)], sem)
            pltpu.async_copy(
                table_hbm.at[idx2.at[pl.ds(k * 128, 128)]],
                cands.at[pl.ds(k * 128, 128)], sem)

    def drain(b):
        for k in range(_NSUB):
            pltpu.make_async_copy(
                table_hbm.at[idx2_b[b].at[pl.ds(k * 128, 128)]],
                rows_b[b].at[pl.ds(k * 128, 128)], sem_b[b]).wait()
            pltpu.make_async_copy(
                table_hbm.at[idx2_b[b].at[pl.ds(k * 128, 128)]],
                cands_b[b].at[pl.ds(k * 128, 128)], sem_b[b]).wait()

    def compute(ci, b):
        rows, cands = rows_b[b], cands_b[b]

        def grp_body(g, _):
            tokv = g * 16 + lane
            acc = jnp.zeros((16,), jnp.float32)
            for j in range(_D):
                jv = lane + j
                jv = jnp.where(jv >= _D, jv - _D, jv)
                a = plsc.load_gather(rows, [tokv, jv])
                c = plsc.load_gather(cands, [tokv, jv])
                acc = acc + a * c
            l = acc * 0.125
            e = jnp.exp(l)
            sv = seg_all[pl.ds(ci * _C + g * 16, 16)]
            plsc.addupdate_scatter(acc1_v, [sv], e)
            plsc.addupdate_scatter(acc2_v, [sv], e * l)
            return 0

        lax.fori_loop(0, _NG, grp_body, 0)

    fire(0, 0)

    def outer(i, _):
        ci0 = i * 2
        for b in (0, 1):
            ci = ci0 + b

            @pl.when(ci + 1 < _CHUNKS)
            def _():
                fire(ci + 1, 1 - b)

            drain(b)
            compute(ci, b)
        return 0

    lax.fori_loop(0, _CHUNKS // 2, outer, 0)

    pltpu.sync_copy(acc1_v, s1_hbm.at[wid])
    pltpu.sync_copy(acc2_v, s2_hbm.at[wid])


_NR = 1000000
_FT = _NR // 128


def _repack_body(tT_hbm, out_hbm, in_0, in_1, in_2, in_3,
                 t_0, t_1, t_2, t_3, tail_in, tail_t,
                 si0, si1, si2, si3, so0, so1, so2, so3):
    wid = lax.axis_index("s") * _NC + lax.axis_index("c")
    nt = jnp.where(wid < 1, 248, 244)
    t0 = jnp.where(wid < 1, 0, 248 + (wid - 1) * 244)

    in_b = (in_0, in_1, in_2, in_3)
    t_b = (t_0, t_1, t_2, t_3)
    si_b = (si0, si1, si2, si3)
    so_b = (so0, so1, so2, so3)

    lane = lax.iota(jnp.int32, 16)

    def fire_in(i, b):
        c0 = (t0 + i) * 128
        pltpu.async_copy(tT_hbm.at[:, pl.ds(c0, 128)], in_b[b], si_b[b])

    def drain_in(b):
        pltpu.make_async_copy(tT_hbm.at[:, pl.ds(0, 128)], in_b[b],
                              si_b[b]).wait()

    rots = []
    rots64 = []
    for k in range(16):
        rot = lane + k
        rot = jnp.where(rot >= 16, rot - 16, rot)
        rots.append(rot)
        rots64.append(rot * 64)
    rowvs = [jb * 16 + lane for jb in range(4)]

    def transpose(b):
        slab, dst = in_b[b], t_b[b]

        def tb_body(tb, _):
            obases = [tb * 1024 + jb * 16 + lane for jb in range(4)]
            for k in range(16):
                colv = tb * 16 + rots[k]
                for jb in range(4):
                    v = plsc.load_gather(slab, [rowvs[jb], colv])
                    plsc.store_scatter(dst, [obases[jb] + rots64[k]], v)
            return 0

        lax.fori_loop(0, 8, tb_body, 0)

    def fire_out(i, b):
        c0 = (t0 + i) * 128
        pltpu.async_copy(t_b[b], out_hbm.at[pl.ds(c0 * 64, 8192)], so_b[b])

    def drain_out(b):
        pltpu.make_async_copy(t_b[b], out_hbm.at[pl.ds(0, 8192)],
                              so_b[b]).wait()

    fire_in(0, 0)
    fire_in(1, 1)
    fire_in(2, 2)

    def outer(io, _):
        for b in (0, 1, 2, 3):
            i = 4 * io + b

            @pl.when(i + 3 < nt)
            def _():
                fire_in(i + 3, (b + 3) % 4)

            drain_in(b)

            @pl.when(i >= 4)
            def _():
                drain_out(b)

            transpose(b)
            fire_out(i, b)
        return 0

    lax.fori_loop(0, nt // 4, outer, 0)
    drain_out(0)
    drain_out(1)
    drain_out(2)
    drain_out(3)

    @pl.when(wid == _NW - 1)
    def _():
        c0 = _FT * 128
        pltpu.sync_copy(tT_hbm.at[:, pl.ds(c0, 64)], tail_in)

        def tail_body(tb, _):
            obases = [tb * 1024 + jb * 16 + lane for jb in range(4)]
            for k in range(16):
                colv = tb * 16 + rots[k]
                for jb in range(4):
                    v = plsc.load_gather(tail_in, [rowvs[jb], colv])
                    plsc.store_scatter(tail_t, [obases[jb] + rots64[k]], v)
            return 0

        lax.fori_loop(0, 4, tail_body, 0)
        pltpu.sync_copy(tail_t, out_hbm.at[pl.ds(c0 * 64, 4096)])


def _repack(table_t):
    mesh = plsc.VectorSubcoreMesh(core_axis_name="c", subcore_axis_name="s",
                                  num_cores=_NC, num_subcores=_NS)
    f = pl.kernel(
        _repack_body,
        out_type=jax.ShapeDtypeStruct((_NR * _D,), jnp.float32),
        mesh=mesh,
        compiler_params=pltpu.CompilerParams(
            needs_layout_passes=False, use_tc_tiling_on_sc=True),
        scratch_types=(
            pltpu.VMEM((_D, 128), jnp.float32),
            pltpu.VMEM((_D, 128), jnp.float32),
            pltpu.VMEM((_D, 128), jnp.float32),
            pltpu.VMEM((_D, 128), jnp.float32),
            pltpu.VMEM((8192,), jnp.float32),
            pltpu.VMEM((8192,), jnp.float32),
            pltpu.VMEM((8192,), jnp.float32),
            pltpu.VMEM((8192,), jnp.float32),
            pltpu.VMEM((_D, 64), jnp.float32),
            pltpu.VMEM((4096,), jnp.float32),
            pltpu.SemaphoreType.DMA,
            pltpu.SemaphoreType.DMA,
            pltpu.SemaphoreType.DMA,
            pltpu.SemaphoreType.DMA,
            pltpu.SemaphoreType.DMA,
            pltpu.SemaphoreType.DMA,
            pltpu.SemaphoreType.DMA,
            pltpu.SemaphoreType.DMA,
        ),
    )
    return f(table_t)


def _sc_main(cand_idx, nb_idx, seg_ids, rel_table):
    mesh = plsc.VectorSubcoreMesh(core_axis_name="c", subcore_axis_name="s",
                                  num_cores=_NC, num_subcores=_NS)
    f = pl.kernel(
        _sc_body,
        out_type=(jax.ShapeDtypeStruct((_NW, _P), jnp.float32),
                  jax.ShapeDtypeStruct((_NW, _P), jnp.float32)),
        mesh=mesh,
        compiler_params=pltpu.CompilerParams(
            needs_layout_passes=False, use_tc_tiling_on_sc=False),
        scratch_types=(
            pltpu.VMEM((_P,), jnp.int32),
            pltpu.VMEM((_TW,), jnp.int32),
            pltpu.VMEM((_TW,), jnp.int32),
            pltpu.VMEM((_C,), jnp.int32),
            pltpu.VMEM((_C,), jnp.int32),
            pltpu.VMEM((_C, _D), jnp.float32),
            pltpu.VMEM((_C, _D), jnp.float32),
            pltpu.VMEM((_C, _D), jnp.float32),
            pltpu.VMEM((_C, _D), jnp.float32),
            pltpu.VMEM((_P,), jnp.float32),
            pltpu.VMEM((_P,), jnp.float32),
            pltpu.SemaphoreType.DMA,
            pltpu.SemaphoreType.DMA,
        ),
    )
    return f(cand_idx, nb_idx, seg_ids, rel_table)


def _tc_combine_body(s1_ref, s2_ref, out_ref):
    s1 = jnp.sum(s1_ref[...], axis=0)
    s2 = jnp.sum(s2_ref[...], axis=0)
    out_ref[...] = 8.0 * s2 / (s1 + 1e-9)


def kernel(triples, neighbour_indices, segment_ids, rel_table):
    n, m, _ = triples.shape
    cand_idx = triples[:, :, 2].reshape(-1).astype(jnp.int32)
    table_lin = _repack(rel_table.T)
    table_rm = table_lin.reshape(_NR, _D)
    s1p, s2p = _sc_main(cand_idx, neighbour_indices.astype(jnp.int32),
                        segment_ids.astype(jnp.int32), table_rm)
    combine = pl.pallas_call(
        _tc_combine_body,
        out_shape=jax.ShapeDtypeStruct((n, m), jnp.float32),
    )
    return combine(s1p.reshape(_NW, n, m), s2p.reshape(_NW, n, m))


# speedup vs baseline: 1.1841x; 1.1841x over previous
import jax
import jax.numpy as jnp
from jax import lax
from jax.experimental import pallas as pl
from jax.experimental.pallas import tpu as pltpu
from jax.experimental.pallas import tpu_sc as plsc

_D = 64
_P = 4096
_T = 524288
_NC, _NS = 2, 16
_NW = _NC * _NS
_TW = _T // _NW
_C = 128
_NSUB = _C // 128
_NG = _C // 16
_CHUNKS = _TW // _C
_CSEG = 768


def _sc_body(cand_idx_hbm, nb_idx_hbm, seg_hbm, table_hbm, s1_hbm, s2_hbm,
             cand_idx_v, nb_all, seg_all, idx2_0, idx2_1,
             rows_0, rows_1, cands_0, cands_1, cseg_v,
             acc1_v, acc2_v, sem0, sem1):
    wid = lax.axis_index("s") * _NC + lax.axis_index("c")
    base = wid * _TW

    idx2_b = (idx2_0, idx2_1)
    rows_b = (rows_0, rows_1)
    cands_b = (cands_0, cands_1)
    sem_b = (sem0, sem1)

    pltpu.sync_copy(cand_idx_hbm, cand_idx_v)
    pltpu.sync_copy(nb_idx_hbm.at[pl.ds(base, _TW)], nb_all)
    pltpu.sync_copy(seg_hbm.at[pl.ds(base, _TW)], seg_all)

    s_first = seg_all[pl.ds(0, 16)][0]
    s_last = seg_all[pl.ds(_TW - 16, 16)][15]
    cbase = jnp.minimum((s_first // 8) * 8, _P - _CSEG)
    fast = (s_last - cbase) < _CSEG
    slow = jnp.logical_not(fast)

    @pl.when(fast)
    def _():
        for k in range(_CSEG // 128):
            pltpu.async_copy(
                table_hbm.at[cand_idx_v.at[pl.ds(cbase + k * 128, 128)]],
                cseg_v.at[pl.ds(k * 128, 128)], sem0)
        for k in range(_CSEG // 128):
            pltpu.make_async_copy(
                table_hbm.at[cand_idx_v.at[pl.ds(k * 128, 128)]],
                cseg_v.at[pl.ds(k * 128, 128)], sem0).wait()

    zeros16 = jnp.zeros((16,), jnp.float32)

    def zero_body(i, _):
        acc1_v[pl.ds(i * 16, 16)] = zeros16
        acc2_v[pl.ds(i * 16, 16)] = zeros16
        return 0

    lax.fori_loop(0, _P // 16, zero_body, 0, unroll=8)

    lane = lax.iota(jnp.int32, 16)

    def fire(ci, b):
        off = ci * _C
        idx2, rows, cands, sem = idx2_b[b], rows_b[b], cands_b[b], sem_b[b]

        def idx_body(g, _):
            sv = seg_all[pl.ds(off + g * 16, 16)]
            idx2[pl.ds(g * 16, 16)] = plsc.load_gather(cand_idx_v, [sv])
            return 0

        lax.fori_loop(0, _NG, idx_body, 0, unroll=_NG)
        for k in range(_NSUB):
            pltpu.async_copy(
                table_hbm.at[nb_all.at[pl.ds(off + k * 128, 128)]],
                rows.at[pl.ds(k * 128, 128)], sem)

        @pl.when(slow)
        def _():
            for k in range(_NSUB):
                pltpu.async_copy(
                    table_hbm.at[idx2.at[pl.ds(k * 128, 128)]],
                    cands.at[pl.ds(k * 128, 128)], sem)

    def drain(b):
        for k in range(_NSUB):
            pltpu.make_async_copy(
                table_hbm.at[idx2_b[b].at[pl.ds(k * 128, 128)]],
                rows_b[b].at[pl.ds(k * 128, 128)], sem_b[b]).wait()

        @pl.when(slow)
        def _():
            for k in range(_NSUB):
                pltpu.make_async_copy(
                    table_hbm.at[idx2_b[b].at[pl.ds(k * 128, 128)]],
                    cands_b[b].at[pl.ds(k * 128, 128)], sem_b[b]).wait()

    def compute(ci, b):
        rows, cands = rows_b[b], cands_b[b]

        def grp_body(g, _):
            tokv = g * 16 + lane
            sv = seg_all[pl.ds(ci * _C + g * 16, 16)]

            @pl.when(fast)
            def _():
                crow = sv - cbase
                acc = jnp.zeros((16,), jnp.float32)
                for j in range(_D):
                    jv = lane + j
                    jv = jnp.where(jv >= _D, jv - _D, jv)
                    a = plsc.load_gather(rows, [tokv, jv])
                    c = plsc.load_gather(cseg_v, [crow, jv])
                    acc = acc + a * c
                l = acc * 0.125
                e = jnp.exp(l)
                plsc.addupdate_scatter(acc1_v, [sv], e)
                plsc.addupdate_scatter(acc2_v, [sv], e * l)

            @pl.when(slow)
            def _():
                acc = jnp.zeros((16,), jnp.float32)
                for j in range(_D):
                    jv = lane + j
                    jv = jnp.where(jv >= _D, jv - _D, jv)
                    a = plsc.load_gather(rows, [tokv, jv])
                    c = plsc.load_gather(cands, [tokv, jv])
                    acc = acc + a * c
                l = acc * 0.125
                e = jnp.exp(l)
                plsc.addupdate_scatter(acc1_v, [sv], e)
                plsc.addupdate_scatter(acc2_v, [sv], e * l)

            return 0

        lax.fori_loop(0, _NG, grp_body, 0)

    fire(0, 0)

    def outer(i, _):
        ci0 = i * 2
        for b in (0, 1):
            ci = ci0 + b

            @pl.when(ci + 1 < _CHUNKS)
            def _():
                fire(ci + 1, 1 - b)

            drain(b)
            compute(ci, b)
        return 0

    lax.fori_loop(0, _CHUNKS // 2, outer, 0)

    pltpu.sync_copy(acc1_v, s1_hbm.at[wid])
    pltpu.sync_copy(acc2_v, s2_hbm.at[wid])


_NR = 1000000
_FT = _NR // 128


def _repack_body(tT_hbm, out_hbm, in_0, in_1, in_2, in_3,
                 t_0, t_1, t_2, t_3, tail_in, tail_t,
                 si0, si1, si2, si3, so0, so1, so2, so3):
    wid = lax.axis_index("s") * _NC + lax.axis_index("c")
    nt = jnp.where(wid < 1, 248, 244)
    t0 = jnp.where(wid < 1, 0, 248 + (wid - 1) * 244)

    in_b = (in_0, in_1, in_2, in_3)
    t_b = (t_0, t_1, t_2, t_3)
    si_b = (si0, si1, si2, si3)
    so_b = (so0, so1, so2, so3)

    lane = lax.iota(jnp.int32, 16)

    def fire_in(i, b):
        c0 = (t0 + i) * 128
        pltpu.async_copy(tT_hbm.at[:, pl.ds(c0, 128)], in_b[b], si_b[b])

    def drain_in(b):
        pltpu.make_async_copy(tT_hbm.at[:, pl.ds(0, 128)], in_b[b],
                              si_b[b]).wait()

    rots = []
    rots64 = []
    for k in range(16):
        rot = lane + k
        rot = jnp.where(rot >= 16, rot - 16, rot)
        rots.append(rot)
        rots64.append(rot * 64)
    rowvs = [jb * 16 + lane for jb in range(4)]

    def transpose(b):
        slab, dst = in_b[b], t_b[b]

        def tb_body(tb, _):
            obases = [tb * 1024 + jb * 16 + lane for jb in range(4)]
            for k in range(16):
                colv = tb * 16 + rots[k]
                for jb in range(4):
                    v = plsc.load_gather(slab, [rowvs[jb], colv])
                    plsc.store_scatter(dst, [obases[jb] + rots64[k]], v)
            return 0

        lax.fori_loop(0, 8, tb_body, 0)

    def fire_out(i, b):
        c0 = (t0 + i) * 128
        pltpu.async_copy(t_b[b], out_hbm.at[pl.ds(c0 * 64, 8192)], so_b[b])

    def drain_out(b):
        pltpu.make_async_copy(t_b[b], out_hbm.at[pl.ds(0, 8192)],
                              so_b[b]).wait()

    fire_in(0, 0)
    fire_in(1, 1)
    fire_in(2, 2)

    def outer(io, _):
        for b in (0, 1, 2, 3):
            i = 4 * io + b

            @pl.when(i + 3 < nt)
            def _():
                fire_in(i + 3, (b + 3) % 4)

            drain_in(b)

            @pl.when(i >= 4)
            def _():
                drain_out(b)

            transpose(b)
            fire_out(i, b)
        return 0

    lax.fori_loop(0, nt // 4, outer, 0)
    drain_out(0)
    drain_out(1)
    drain_out(2)
    drain_out(3)

    @pl.when(wid == _NW - 1)
    def _():
        c0 = _FT * 128
        pltpu.sync_copy(tT_hbm.at[:, pl.ds(c0, 64)], tail_in)

        def tail_body(tb, _):
            obases = [tb * 1024 + jb * 16 + lane for jb in range(4)]
            for k in range(16):
                colv = tb * 16 + rots[k]
                for jb in range(4):
                    v = plsc.load_gather(tail_in, [rowvs[jb], colv])
                    plsc.store_scatter(tail_t, [obases[jb] + rots64[k]], v)
            return 0

        lax.fori_loop(0, 4, tail_body, 0)
        pltpu.sync_copy(tail_t, out_hbm.at[pl.ds(c0 * 64, 4096)])


def _repack(table_t):
    mesh = plsc.VectorSubcoreMesh(core_axis_name="c", subcore_axis_name="s",
                                  num_cores=_NC, num_subcores=_NS)
    f = pl.kernel(
        _repack_body,
        out_type=jax.ShapeDtypeStruct((_NR * _D,), jnp.float32),
        mesh=mesh,
        compiler_params=pltpu.CompilerParams(
            needs_layout_passes=False, use_tc_tiling_on_sc=True),
        scratch_types=(
            pltpu.VMEM((_D, 128), jnp.float32),
            pltpu.VMEM((_D, 128), jnp.float32),
            pltpu.VMEM((_D, 128), jnp.float32),
            pltpu.VMEM((_D, 128), jnp.float32),
            pltpu.VMEM((8192,), jnp.float32),
            pltpu.VMEM((8192,), jnp.float32),
            pltpu.VMEM((8192,), jnp.float32),
            pltpu.VMEM((8192,), jnp.float32),
            pltpu.VMEM((_D, 64), jnp.float32),
            pltpu.VMEM((4096,), jnp.float32),
            pltpu.SemaphoreType.DMA,
            pltpu.SemaphoreType.DMA,
            pltpu.SemaphoreType.DMA,
            pltpu.SemaphoreType.DMA,
            pltpu.SemaphoreType.DMA,
            pltpu.SemaphoreType.DMA,
            pltpu.SemaphoreType.DMA,
            pltpu.SemaphoreType.DMA,
        ),
    )
    return f(table_t)


def _sc_main(cand_idx, nb_idx, seg_ids, rel_table):
    mesh = plsc.VectorSubcoreMesh(core_axis_name="c", subcore_axis_name="s",
                                  num_cores=_NC, num_subcores=_NS)
    f = pl.kernel(
        _sc_body,
        out_type=(jax.ShapeDtypeStruct((_NW, _P), jnp.float32),
                  jax.ShapeDtypeStruct((_NW, _P), jnp.float32)),
        mesh=mesh,
        compiler_params=pltpu.CompilerParams(
            needs_layout_passes=False, use_tc_tiling_on_sc=False),
        scratch_types=(
            pltpu.VMEM((_P,), jnp.int32),
            pltpu.VMEM((_TW,), jnp.int32),
            pltpu.VMEM((_TW,), jnp.int32),
            pltpu.VMEM((_C,), jnp.int32),
            pltpu.VMEM((_C,), jnp.int32),
            pltpu.VMEM((_C, _D), jnp.float32),
            pltpu.VMEM((_C, _D), jnp.float32),
            pltpu.VMEM((_C, _D), jnp.float32),
            pltpu.VMEM((_C, _D), jnp.float32),
            pltpu.VMEM((_CSEG, _D), jnp.float32),
            pltpu.VMEM((_P,), jnp.float32),
            pltpu.VMEM((_P,), jnp.float32),
            pltpu.SemaphoreType.DMA,
            pltpu.SemaphoreType.DMA,
        ),
    )
    return f(cand_idx, nb_idx, seg_ids, rel_table)


def _tc_combine_body(s1_ref, s2_ref, out_ref):
    s1 = jnp.sum(s1_ref[...], axis=0)
    s2 = jnp.sum(s2_ref[...], axis=0)
    out_ref[...] = 8.0 * s2 / (s1 + 1e-9)


def kernel(triples, neighbour_indices, segment_ids, rel_table):
    n, m, _ = triples.shape
    cand_idx = triples[:, :, 2].reshape(-1).astype(jnp.int32)
    table_lin = _repack(rel_table.T)
    table_rm = table_lin.reshape(_NR, _D)
    s1p, s2p = _sc_main(cand_idx, neighbour_indices.astype(jnp.int32),
                        segment_ids.astype(jnp.int32), table_rm)
    combine = pl.pallas_call(
        _tc_combine_body,
        out_shape=jax.ShapeDtypeStruct((n, m), jnp.float32),
    )
    return combine(s1p.reshape(_NW, n, m), s2p.reshape(_NW, n, m))


# speedup vs baseline: 1.3233x; 1.1175x over previous
import jax
import jax.numpy as jnp
from jax import lax
from jax.experimental import pallas as pl
from jax.experimental.pallas import tpu as pltpu
from jax.experimental.pallas import tpu_sc as plsc

_D = 64
_P = 4096
_T = 524288
_NC, _NS = 2, 16
_NW = _NC * _NS
_TW = _T // _NW
_C = 128
_NSUB = _C // 128
_NG = _C // 16
_CHUNKS = _TW // _C
_CSEG = 768


def _sc_body(cand_idx_hbm, nb_idx_hbm, seg_hbm, table_hbm, s1_hbm, s2_hbm,
             cand_idx_v, nb_all, seg_all, idx2_0, idx2_1,
             rows_0, rows_1, cands_0, cands_1, cseg_v,
             acc1_v, acc2_v, sem0, sem1):
    wid = lax.axis_index("s") * _NC + lax.axis_index("c")
    base = wid * _TW

    idx2_b = (idx2_0, idx2_1)
    rows_b = (rows_0, rows_1)
    cands_b = (cands_0, cands_1)
    sem_b = (sem0, sem1)

    pltpu.sync_copy(cand_idx_hbm, cand_idx_v)
    pltpu.sync_copy(nb_idx_hbm.at[pl.ds(base, _TW)], nb_all)
    pltpu.sync_copy(seg_hbm.at[pl.ds(base, _TW)], seg_all)

    s_first = seg_all[pl.ds(0, 16)][0]
    s_last = seg_all[pl.ds(_TW - 16, 16)][15]
    cbase = jnp.minimum((s_first // 8) * 8, _P - _CSEG)
    fast = (s_last - cbase) < _CSEG
    slow = jnp.logical_not(fast)

    @pl.when(fast)
    def _():
        for k in range(_CSEG // 128):
            pltpu.async_copy(
                table_hbm.at[cand_idx_v.at[pl.ds(cbase + k * 128, 128)]],
                cseg_v.at[pl.ds(k * 128, 128)], sem0)
        for k in range(_CSEG // 128):
            pltpu.make_async_copy(
                table_hbm.at[cand_idx_v.at[pl.ds(k * 128, 128)]],
                cseg_v.at[pl.ds(k * 128, 128)], sem0).wait()

    zeros16 = jnp.zeros((16,), jnp.float32)

    def zero_body(i, _):
        acc1_v[pl.ds(i * 16, 16)] = zeros16
        acc2_v[pl.ds(i * 16, 16)] = zeros16
        return 0

    lax.fori_loop(0, _P // 16, zero_body, 0, unroll=8)

    lane = lax.iota(jnp.int32, 16)

    def fire(ci, b):
        off = ci * _C
        idx2, rows, cands, sem = idx2_b[b], rows_b[b], cands_b[b], sem_b[b]

        def idx_body(g, _):
            sv = seg_all[pl.ds(off + g * 16, 16)]
            idx2[pl.ds(g * 16, 16)] = plsc.load_gather(cand_idx_v, [sv])
            return 0

        lax.fori_loop(0, _NG, idx_body, 0, unroll=_NG)
        for k in range(_NSUB):
            pltpu.async_copy(
                table_hbm.at[nb_all.at[pl.ds(off + k * 128, 128)]],
                rows.at[pl.ds(k * 128, 128)], sem)

        @pl.when(slow)
        def _():
            for k in range(_NSUB):
                pltpu.async_copy(
                    table_hbm.at[idx2.at[pl.ds(k * 128, 128)]],
                    cands.at[pl.ds(k * 128, 128)], sem)

    def drain(b):
        for k in range(_NSUB):
            pltpu.make_async_copy(
                table_hbm.at[idx2_b[b].at[pl.ds(k * 128, 128)]],
                rows_b[b].at[pl.ds(k * 128, 128)], sem_b[b]).wait()

        @pl.when(slow)
        def _():
            for k in range(_NSUB):
                pltpu.make_async_copy(
                    table_hbm.at[idx2_b[b].at[pl.ds(k * 128, 128)]],
                    cands_b[b].at[pl.ds(k * 128, 128)], sem_b[b]).wait()

    def compute(ci, b):
        rows, cands = rows_b[b], cands_b[b]

        def grp_body(g, _):
            tokv = g * 16 + lane
            sv = seg_all[pl.ds(ci * _C + g * 16, 16)]

            @pl.when(fast)
            def _():
                crow = sv - cbase
                acc = jnp.zeros((16,), jnp.float32)
                for j in range(_D):
                    jv = lane + j
                    jv = jnp.where(jv >= _D, jv - _D, jv)
                    a = plsc.load_gather(rows, [tokv, jv])
                    c = plsc.load_gather(cseg_v, [crow, jv])
                    acc = acc + a * c
                l = acc * 0.125
                e = jnp.exp(l)
                plsc.addupdate_scatter(acc1_v, [sv], e)
                plsc.addupdate_scatter(acc2_v, [sv], e * l)

            @pl.when(slow)
            def _():
                acc = jnp.zeros((16,), jnp.float32)
                for j in range(_D):
                    jv = lane + j
                    jv = jnp.where(jv >= _D, jv - _D, jv)
                    a = plsc.load_gather(rows, [tokv, jv])
                    c = plsc.load_gather(cands, [tokv, jv])
                    acc = acc + a * c
                l = acc * 0.125
                e = jnp.exp(l)
                plsc.addupdate_scatter(acc1_v, [sv], e)
                plsc.addupdate_scatter(acc2_v, [sv], e * l)

            return 0

        lax.fori_loop(0, _NG, grp_body, 0)

    fire(0, 0)

    def outer(i, _):
        ci0 = i * 2
        for b in (0, 1):
            ci = ci0 + b

            @pl.when(ci + 1 < _CHUNKS)
            def _():
                fire(ci + 1, 1 - b)

            drain(b)
            compute(ci, b)
        return 0

    lax.fori_loop(0, _CHUNKS // 2, outer, 0)

    pltpu.sync_copy(acc1_v, s1_hbm.at[wid])
    pltpu.sync_copy(acc2_v, s2_hbm.at[wid])


_NR = 1000000
_FT = _NR // 128


def _repack_body(tT_hbm, out_hbm, in_0, in_1, t_0, t_1, tail_in, tail_t,
                 si0, si1, so0, so1):
    wid = lax.axis_index("s") * _NC + lax.axis_index("c")
    nt = jnp.where(wid < 1, 124, 122)
    t0 = jnp.where(wid < 1, 0, 124 + (wid - 1) * 122)

    in_b = (in_0, in_1)
    t_b = (t_0, t_1)
    si_b = (si0, si1)
    so_b = (so0, so1)

    lane = lax.iota(jnp.int32, 16)

    def fire_in(i, b):
        c0 = (t0 + i) * 256
        pltpu.async_copy(tT_hbm.at[:, pl.ds(c0, 256)], in_b[b], si_b[b])

    def drain_in(b):
        pltpu.make_async_copy(tT_hbm.at[:, pl.ds(0, 256)], in_b[b],
                              si_b[b]).wait()

    rots = []
    rots64 = []
    for k in range(16):
        rot = lane + k
        rot = jnp.where(rot >= 16, rot - 16, rot)
        rots.append(rot)
        rots64.append(rot * 64)
    rowvs = [jb * 16 + lane for jb in range(4)]

    def transpose(b):
        slab, dst = in_b[b], t_b[b]

        def tb_body(tb, _):
            obases = [tb * 1024 + jb * 16 + lane for jb in range(4)]
            for k in range(16):
                colv = tb * 16 + rots[k]
                for jb in range(4):
                    v = plsc.load_gather(slab, [rowvs[jb], colv])
                    plsc.store_scatter(dst, [obases[jb] + rots64[k]], v)
            return 0

        lax.fori_loop(0, 16, tb_body, 0)

    def fire_out(i, b):
        c0 = (t0 + i) * 256
        pltpu.async_copy(t_b[b], out_hbm.at[pl.ds(c0 * 64, 16384)], so_b[b])

    def drain_out(b):
        pltpu.make_async_copy(t_b[b], out_hbm.at[pl.ds(0, 16384)],
                              so_b[b]).wait()

    fire_in(0, 0)

    def outer(io, _):
        for b in (0, 1):
            i = 2 * io + b

            @pl.when(i + 1 < nt)
            def _():
                fire_in(i + 1, 1 - b)

            drain_in(b)

            @pl.when(i >= 2)
            def _():
                drain_out(b)

            transpose(b)
            fire_out(i, b)
        return 0

    lax.fori_loop(0, nt // 2, outer, 0)
    drain_out(0)
    drain_out(1)

    @pl.when(wid == _NW - 1)
    def _():
        c0 = _FT * 128
        pltpu.sync_copy(tT_hbm.at[:, pl.ds(c0, 64)], tail_in)

        def tail_body(tb, _):
            obases = [tb * 1024 + jb * 16 + lane for jb in range(4)]
            for k in range(16):
                colv = tb * 16 + rots[k]
                for jb in range(4):
                    v = plsc.load_gather(tail_in, [rowvs[jb], colv])
                    plsc.store_scatter(tail_t, [obases[jb] + rots64[k]], v)
            return 0

        lax.fori_loop(0, 4, tail_body, 0)
        pltpu.sync_copy(tail_t, out_hbm.at[pl.ds(c0 * 64, 4096)])


def _repack(table_t):
    mesh = plsc.VectorSubcoreMesh(core_axis_name="c", subcore_axis_name="s",
                                  num_cores=_NC, num_subcores=_NS)
    f = pl.kernel(
        _repack_body,
        out_type=jax.ShapeDtypeStruct((_NR * _D,), jnp.float32),
        mesh=mesh,
        compiler_params=pltpu.CompilerParams(
            needs_layout_passes=False, use_tc_tiling_on_sc=True),
        scratch_types=(
            pltpu.VMEM((_D, 256), jnp.float32),
            pltpu.VMEM((_D, 256), jnp.float32),
            pltpu.VMEM((16384,), jnp.float32),
            pltpu.VMEM((16384,), jnp.float32),
            pltpu.VMEM((_D, 64), jnp.float32),
            pltpu.VMEM((4096,), jnp.float32),
            pltpu.SemaphoreType.DMA,
            pltpu.SemaphoreType.DMA,
            pltpu.SemaphoreType.DMA,
            pltpu.SemaphoreType.DMA,
        ),
    )
    return f(table_t)


def _sc_main(cand_idx, nb_idx, seg_ids, rel_table):
    mesh = plsc.VectorSubcoreMesh(core_axis_name="c", subcore_axis_name="s",
                                  num_cores=_NC, num_subcores=_NS)
    f = pl.kernel(
        _sc_body,
        out_type=(jax.ShapeDtypeStruct((_NW, _P), jnp.float32),
                  jax.ShapeDtypeStruct((_NW, _P), jnp.float32)),
        mesh=mesh,
        compiler_params=pltpu.CompilerParams(
            needs_layout_passes=False, use_tc_tiling_on_sc=False),
        scratch_types=(
            pltpu.VMEM((_P,), jnp.int32),
            pltpu.VMEM((_TW,), jnp.int32),
            pltpu.VMEM((_TW,), jnp.int32),
            pltpu.VMEM((_C,), jnp.int32),
            pltpu.VMEM((_C,), jnp.int32),
            pltpu.VMEM((_C, _D), jnp.float32),
            pltpu.VMEM((_C, _D), jnp.float32),
            pltpu.VMEM((_C, _D), jnp.float32),
            pltpu.VMEM((_C, _D), jnp.float32),
            pltpu.VMEM((_CSEG, _D), jnp.float32),
            pltpu.VMEM((_P,), jnp.float32),
            pltpu.VMEM((_P,), jnp.float32),
            pltpu.SemaphoreType.DMA,
            pltpu.SemaphoreType.DMA,
        ),
    )
    return f(cand_idx, nb_idx, seg_ids, rel_table)


def _tc_combine_body(s1_ref, s2_ref, out_ref):
    s1 = jnp.sum(s1_ref[...], axis=0)
    s2 = jnp.sum(s2_ref[...], axis=0)
    out_ref[...] = 8.0 * s2 / (s1 + 1e-9)


def kernel(triples, neighbour_indices, segment_ids, rel_table):
    n, m, _ = triples.shape
    cand_idx = triples[:, :, 2].reshape(-1).astype(jnp.int32)
    table_lin = _repack(rel_table.T)
    table_rm = table_lin.reshape(_NR, _D)
    s1p, s2p = _sc_main(cand_idx, neighbour_indices.astype(jnp.int32),
                        segment_ids.astype(jnp.int32), table_rm)
    combine = pl.pallas_call(
        _tc_combine_body,
        out_shape=jax.ShapeDtypeStruct((n, m), jnp.float32),
    )
    return combine(s1p.reshape(_NW, n, m), s2p.reshape(_NW, n, m))


# speedup vs baseline: 1.3247x; 1.0011x over previous
import jax
import jax.numpy as jnp
from jax import lax
from jax.experimental import pallas as pl
from jax.experimental.pallas import tpu as pltpu
from jax.experimental.pallas import tpu_sc as plsc

_D = 64
_P = 4096
_T = 524288
_NC, _NS = 2, 16
_NW = _NC * _NS
_TW = _T // _NW
_C = 128
_NSUB = _C // 128
_NG = _C // 16
_CHUNKS = _TW // _C
_CSEG = 768


def _sc_body(cand_idx_hbm, nb_idx_hbm, seg_hbm, table_hbm, s1_hbm, s2_hbm,
             cand_idx_v, nb_all, seg_all, idx2_0, idx2_1,
             rows_0, rows_1, cands_0, cands_1, cseg_v,
             acc1_v, acc2_v, sem0, sem1):
    wid = lax.axis_index("s") * _NC + lax.axis_index("c")
    base = wid * _TW

    idx2_b = (idx2_0, idx2_1)
    rows_b = (rows_0, rows_1)
    cands_b = (cands_0, cands_1)
    sem_b = (sem0, sem1)

    pltpu.sync_copy(cand_idx_hbm, cand_idx_v)
    pltpu.sync_copy(nb_idx_hbm.at[pl.ds(base, _TW)], nb_all)
    pltpu.sync_copy(seg_hbm.at[pl.ds(base, _TW)], seg_all)

    s_first = seg_all[pl.ds(0, 16)][0]
    s_last = seg_all[pl.ds(_TW - 16, 16)][15]
    cbase = jnp.minimum((s_first // 8) * 8, _P - _CSEG)
    fast = (s_last - cbase) < _CSEG
    slow = jnp.logical_not(fast)

    @pl.when(fast)
    def _():
        for k in range(_CSEG // 128):
            pltpu.async_copy(
                table_hbm.at[cand_idx_v.at[pl.ds(cbase + k * 128, 128)]],
                cseg_v.at[pl.ds(k * 128, 128)], sem0)
        for k in range(_CSEG // 128):
            pltpu.make_async_copy(
                table_hbm.at[cand_idx_v.at[pl.ds(k * 128, 128)]],
                cseg_v.at[pl.ds(k * 128, 128)], sem0).wait()

    zeros16 = jnp.zeros((16,), jnp.float32)

    def zero_body(i, _):
        acc1_v[pl.ds(i * 16, 16)] = zeros16
        acc2_v[pl.ds(i * 16, 16)] = zeros16
        return 0

    lax.fori_loop(0, _P // 16, zero_body, 0, unroll=8)

    lane = lax.iota(jnp.int32, 16)

    def fire(ci, b):
        off = ci * _C
        idx2, rows, cands, sem = idx2_b[b], rows_b[b], cands_b[b], sem_b[b]

        def idx_body(g, _):
            sv = seg_all[pl.ds(off + g * 16, 16)]
            idx2[pl.ds(g * 16, 16)] = plsc.load_gather(cand_idx_v, [sv])
            return 0

        lax.fori_loop(0, _NG, idx_body, 0, unroll=_NG)
        for k in range(_NSUB):
            pltpu.async_copy(
                table_hbm.at[nb_all.at[pl.ds(off + k * 128, 128)]],
                rows.at[pl.ds(k * 128, 128)], sem)

        @pl.when(slow)
        def _():
            for k in range(_NSUB):
                pltpu.async_copy(
                    table_hbm.at[idx2.at[pl.ds(k * 128, 128)]],
                    cands.at[pl.ds(k * 128, 128)], sem)

    def drain(b):
        for k in range(_NSUB):
            pltpu.make_async_copy(
                table_hbm.at[idx2_b[b].at[pl.ds(k * 128, 128)]],
                rows_b[b].at[pl.ds(k * 128, 128)], sem_b[b]).wait()

        @pl.when(slow)
        def _():
            for k in range(_NSUB):
                pltpu.make_async_copy(
                    table_hbm.at[idx2_b[b].at[pl.ds(k * 128, 128)]],
                    cands_b[b].at[pl.ds(k * 128, 128)], sem_b[b]).wait()

    def compute(ci, b):
        rows, cands = rows_b[b], cands_b[b]

        def grp_body(g, _):
            tokv = g * 16 + lane
            sv = seg_all[pl.ds(ci * _C + g * 16, 16)]

            @pl.when(fast)
            def _():
                crow = sv - cbase
                acc = jnp.zeros((16,), jnp.float32)
                for j in range(_D):
                    jv = lane + j
                    jv = jnp.where(jv >= _D, jv - _D, jv)
                    a = plsc.load_gather(rows, [tokv, jv])
                    c = plsc.load_gather(cseg_v, [crow, jv])
                    acc = acc + a * c
                l = acc * 0.125
                e = jnp.exp(l)
                plsc.addupdate_scatter(acc1_v, [sv], e)
                plsc.addupdate_scatter(acc2_v, [sv], e * l)

            @pl.when(slow)
            def _():
                acc = jnp.zeros((16,), jnp.float32)
                for j in range(_D):
                    jv = lane + j
                    jv = jnp.where(jv >= _D, jv - _D, jv)
                    a = plsc.load_gather(rows, [tokv, jv])
                    c = plsc.load_gather(cands, [tokv, jv])
                    acc = acc + a * c
                l = acc * 0.125
                e = jnp.exp(l)
                plsc.addupdate_scatter(acc1_v, [sv], e)
                plsc.addupdate_scatter(acc2_v, [sv], e * l)

            return 0

        lax.fori_loop(0, _NG, grp_body, 0)

    fire(0, 0)

    def outer(i, _):
        ci0 = i * 2
        for b in (0, 1):
            ci = ci0 + b

            @pl.when(ci + 1 < _CHUNKS)
            def _():
                fire(ci + 1, 1 - b)

            drain(b)
            compute(ci, b)
        return 0

    lax.fori_loop(0, _CHUNKS // 2, outer, 0)

    pltpu.sync_copy(acc1_v, s1_hbm.at[wid])
    pltpu.sync_copy(acc2_v, s2_hbm.at[wid])


_NR = 1000000
_FT = _NR // 128


def _repack_body(tT_hbm, out_hbm, in_0, in_1, t_0, t_1, tail_in, tail_t,
                 si0, si1, so0, so1):
    wid = lax.axis_index("s") * _NC + lax.axis_index("c")
    nt = jnp.where(wid < 22, 82, 80)
    t0 = jnp.where(wid < 22, wid * 82, 22 * 82 + (wid - 22) * 80)

    in_b = (in_0, in_1)
    t_b = (t_0, t_1)
    si_b = (si0, si1)
    so_b = (so0, so1)

    lane = lax.iota(jnp.int32, 16)

    def fire_in(i, b):
        c0 = (t0 + i) * 384
        pltpu.async_copy(tT_hbm.at[:, pl.ds(c0, 384)], in_b[b], si_b[b])

    def drain_in(b):
        pltpu.make_async_copy(tT_hbm.at[:, pl.ds(0, 384)], in_b[b],
                              si_b[b]).wait()

    rots = []
    rots64 = []
    for k in range(16):
        rot = lane + k
        rot = jnp.where(rot >= 16, rot - 16, rot)
        rots.append(rot)
        rots64.append(rot * 64)
    rowvs = [jb * 16 + lane for jb in range(4)]

    def transpose(b):
        slab, dst = in_b[b], t_b[b]

        def tb_body(tb, _):
            obases = [tb * 1024 + jb * 16 + lane for jb in range(4)]
            for k in range(16):
                colv = tb * 16 + rots[k]
                for jb in range(4):
                    v = plsc.load_gather(slab, [rowvs[jb], colv])
                    plsc.store_scatter(dst, [obases[jb] + rots64[k]], v)
            return 0

        lax.fori_loop(0, 24, tb_body, 0)

    def fire_out(i, b):
        c0 = (t0 + i) * 384
        pltpu.async_copy(t_b[b], out_hbm.at[pl.ds(c0 * 64, 24576)], so_b[b])

    def drain_out(b):
        pltpu.make_async_copy(t_b[b], out_hbm.at[pl.ds(0, 24576)],
                              so_b[b]).wait()

    fire_in(0, 0)

    def outer(io, _):
        for b in (0, 1):
            i = 2 * io + b

            @pl.when(i + 1 < nt)
            def _():
                fire_in(i + 1, 1 - b)

            drain_in(b)

            @pl.when(i >= 2)
            def _():
                drain_out(b)

            transpose(b)
            fire_out(i, b)
        return 0

    lax.fori_loop(0, nt // 2, outer, 0)
    drain_out(0)
    drain_out(1)

    @pl.when(wid == _NW - 1)
    def _():
        c0 = _FT * 128
        pltpu.sync_copy(tT_hbm.at[:, pl.ds(c0, 64)], tail_in)

        def tail_body(tb, _):
            obases = [tb * 1024 + jb * 16 + lane for jb in range(4)]
            for k in range(16):
                colv = tb * 16 + rots[k]
                for jb in range(4):
                    v = plsc.load_gather(tail_in, [rowvs[jb], colv])
                    plsc.store_scatter(tail_t, [obases[jb] + rots64[k]], v)
            return 0

        lax.fori_loop(0, 4, tail_body, 0)
        pltpu.sync_copy(tail_t, out_hbm.at[pl.ds(c0 * 64, 4096)])


def _repack(table_t):
    mesh = plsc.VectorSubcoreMesh(core_axis_name="c", subcore_axis_name="s",
                                  num_cores=_NC, num_subcores=_NS)
    f = pl.kernel(
        _repack_body,
        out_type=jax.ShapeDtypeStruct((_NR * _D,), jnp.float32),
        mesh=mesh,
        compiler_params=pltpu.CompilerParams(
            needs_layout_passes=False, use_tc_tiling_on_sc=True),
        scratch_types=(
            pltpu.VMEM((_D, 384), jnp.float32),
            pltpu.VMEM((_D, 384), jnp.float32),
            pltpu.VMEM((24576,), jnp.float32),
            pltpu.VMEM((24576,), jnp.float32),
            pltpu.VMEM((_D, 64), jnp.float32),
            pltpu.VMEM((4096,), jnp.float32),
            pltpu.SemaphoreType.DMA,
            pltpu.SemaphoreType.DMA,
            pltpu.SemaphoreType.DMA,
            pltpu.SemaphoreType.DMA,
        ),
    )
    return f(table_t)


def _sc_main(cand_idx, nb_idx, seg_ids, rel_table):
    mesh = plsc.VectorSubcoreMesh(core_axis_name="c", subcore_axis_name="s",
                                  num_cores=_NC, num_subcores=_NS)
    f = pl.kernel(
        _sc_body,
        out_type=(jax.ShapeDtypeStruct((_NW, _P), jnp.float32),
                  jax.ShapeDtypeStruct((_NW, _P), jnp.float32)),
        mesh=mesh,
        compiler_params=pltpu.CompilerParams(
            needs_layout_passes=False, use_tc_tiling_on_sc=False),
        scratch_types=(
            pltpu.VMEM((_P,), jnp.int32),
            pltpu.VMEM((_TW,), jnp.int32),
            pltpu.VMEM((_TW,), jnp.int32),
            pltpu.VMEM((_C,), jnp.int32),
            pltpu.VMEM((_C,), jnp.int32),
            pltpu.VMEM((_C, _D), jnp.float32),
            pltpu.VMEM((_C, _D), jnp.float32),
            pltpu.VMEM((_C, _D), jnp.float32),
            pltpu.VMEM((_C, _D), jnp.float32),
            pltpu.VMEM((_CSEG, _D), jnp.float32),
            pltpu.VMEM((_P,), jnp.float32),
            pltpu.VMEM((_P,), jnp.float32),
            pltpu.SemaphoreType.DMA,
            pltpu.SemaphoreType.DMA,
        ),
    )
    return f(cand_idx, nb_idx, seg_ids, rel_table)


def _tc_combine_body(s1_ref, s2_ref, out_ref):
    s1 = jnp.sum(s1_ref[...], axis=0)
    s2 = jnp.sum(s2_ref[...], axis=0)
    out_ref[...] = 8.0 * s2 / (s1 + 1e-9)


def kernel(triples, neighbour_indices, segment_ids, rel_table):
    n, m, _ = triples.shape
    cand_idx = triples[:, :, 2].reshape(-1).astype(jnp.int32)
    table_lin = _repack(rel_table.T)
    table_rm = table_lin.reshape(_NR, _D)
    s1p, s2p = _sc_main(cand_idx, neighbour_indices.astype(jnp.int32),
                        segment_ids.astype(jnp.int32), table_rm)
    combine = pl.pallas_call(
        _tc_combine_body,
        out_shape=jax.ShapeDtypeStruct((n, m), jnp.float32),
    )
    return combine(s1p.reshape(_NW, n, m), s2p.reshape(_NW, n, m))
